# baseline (speedup 1.0000x reference)
import jax
import jax.numpy as jnp
from jax import lax
from jax.experimental import pallas as pl
from jax.experimental.pallas import tpu as pltpu
from jax.experimental.pallas import tpu_sc as plsc
from jax._src.pallas import mpmd as _pl_mpmd
from jax._src.pallas.mosaic import core as _tpu_core

_LANES = 16
_NUM_SUBCORES = 8
_B = 4096
_BPW = _B // _NUM_SUBCORES
_TABLE = 1001

_scs_mesh = plsc.ScalarSubcoreMesh(axis_name="c", num_cores=1)
_tec_mesh = plsc.VectorSubcoreMesh(
    core_axis_name="c", subcore_axis_name="s", num_cores=1,
    num_subcores=_NUM_SUBCORES,
)
def _scs_fn(betas_hbm, t_hbm, out_hbm, table_sh, idx_sh, rdy_i, rdy_t, sem_scs):
    idx_cp = pltpu.async_copy(t_hbm, idx_sh, sem_scs)
    tbl_cp = pltpu.async_copy(betas_hbm, table_sh, sem_scs)
    idx_cp.wait()
    for i in range(_NUM_SUBCORES):
        pl.semaphore_signal(rdy_i, 1, device_id={"s": i})
    tbl_cp.wait()
    for i in range(_NUM_SUBCORES):
        pl.semaphore_signal(rdy_t, 1, device_id={"s": i})


def _tec_fn(betas_hbm, t_hbm, out_hbm, table_sh, idx_sh, rdy_i, rdy_t, sem_scs):
    wid = lax.axis_index("s")
    base = wid * _BPW

    def body(table_v, idx_v, out_v, sem_tec):
        pl.semaphore_wait(rdy_i, 1)
        idx_cp = pltpu.async_copy(idx_sh.at[pl.ds(base, _BPW)], idx_v, sem_tec)
        pl.semaphore_wait(rdy_t, 1)
        tbl_cp = pltpu.async_copy(table_sh, table_v, sem_tec)
        idx_cp.wait()
        tbl_cp.wait()
        for j in range(_BPW // _LANES):
            idx = idx_v[pl.ds(j * _LANES, _LANES)]
            out_v[pl.ds(j * _LANES, _LANES)] = plsc.load_gather(table_v, [idx])
        pltpu.sync_copy(out_v, out_hbm.at[pl.ds(base, _BPW)])

    pl.run_scoped(
        body,
        pltpu.VMEM((_TABLE,), jnp.float32),
        pltpu.VMEM((_BPW,), jnp.int32),
        pltpu.VMEM((_BPW,), jnp.float32),
        pltpu.SemaphoreType.DMA,
    )


_gather_sc = _pl_mpmd.mpmd_map(
    [(_scs_mesh, _scs_fn), (_tec_mesh, _tec_fn)],
    out_types=[jax.ShapeDtypeStruct((_B,), jnp.float32)],
    scratch_types=[
        _tpu_core.MemorySpace.VMEM_SHARED((_TABLE,), jnp.float32),
        _tpu_core.MemorySpace.VMEM_SHARED((_B,), jnp.int32),
        pltpu.SemaphoreType.REGULAR @ _tec_mesh,
        pltpu.SemaphoreType.REGULAR @ _tec_mesh,
        pltpu.SemaphoreType.DMA @ _scs_mesh,
    ],
    compiler_params=pltpu.CompilerParams(needs_layout_passes=False),
)


def kernel(betas, t_int):
    (out,) = _gather_sc(betas.astype(jnp.float32), t_int.astype(jnp.int32))
    return out
